# baseline (speedup 1.0000x reference)
import functools

import jax
import jax.numpy as jnp
from jax import lax
from jax.experimental import pallas as pl
from jax.experimental.pallas import tpu as pltpu
from jax.experimental.pallas import tpu_sc as plsc

_B, _T, _D, _K = 8, 1024, 256, 8192
_NTOK = _B * _T
_TT = 256
_NT = _NTOK // _TT
_CBT = 1024
_NW = 32
_BPW = _NTOK // _NW


_ATT = 256
_ANT = _NTOK // _ATT


def _argmin_body(scale_ref, z_ref, emb_ref, pw_ref, pb_ref,
                 idx_ref, qcb_ref, cbn_ref):
    i = pl.program_id(0)

    @pl.when(i == 0)
    def _():
        for t in range(_K // _CBT):
            q = lax.dot_general(emb_ref[pl.ds(t * _CBT, _CBT), :],
                                pw_ref[...], (((1,), (1,)), ((), ())),
                                preferred_element_type=jnp.float32)
            q = q + pb_ref[...]
            nrm = jnp.sqrt(jnp.sum(q * q, axis=-1, keepdims=True))
            qcb_ref[pl.ds(t * _CBT, _CBT), :] = q
            cbn_ref[pl.ds(t * _CBT, _CBT), :] = q / jnp.maximum(nrm, 1e-12)

    zt = z_ref[...]
    nrm = jnp.sqrt(jnp.sum(zt * zt, axis=-1, keepdims=True))
    zn = zt / jnp.maximum(nrm, 1e-12)
    s = lax.dot_general(zn, cbn_ref[...],
                        (((1,), (1,)), ((), ())),
                        preferred_element_type=jnp.float32)
    d = s * (-scale_ref[0])
    idx_ref[0, 0, :] = jnp.argmin(d, axis=1).astype(jnp.int32)


def _argmin_distances(z_flat, emb_w, proj_w, proj_b, scale):
    idx3, qcb = pl.pallas_call(
        _argmin_body,
        grid=(_ANT,),
        in_specs=[
            pl.BlockSpec(memory_space=pltpu.SMEM),
            pl.BlockSpec((_ATT, _D), lambda i: (i, 0)),
            pl.BlockSpec((_K, _D), lambda i: (0, 0)),
            pl.BlockSpec((_D, _D), lambda i: (0, 0)),
            pl.BlockSpec((1, _D), lambda i: (0, 0)),
        ],
        out_specs=[
            pl.BlockSpec((1, 1, _ATT), lambda i: (i, 0, 0)),
            pl.BlockSpec((_K, _D), lambda i: (0, 0)),
        ],
        out_shape=[
            jax.ShapeDtypeStruct((_ANT, 1, _ATT), jnp.int32),
            jax.ShapeDtypeStruct((_K, _D), jnp.float32),
        ],
        scratch_shapes=[pltpu.VMEM((_K, _D), jnp.float32)],
    )(scale.reshape(1), z_flat, emb_w, proj_w, proj_b.reshape(1, _D))
    return idx3.reshape(_NTOK), qcb


_ZCH = 64
_NZC = _BPW // _ZCH
_NLV = _D // 16


@functools.cache
def _make_gather_sc():
    @functools.partial(
        pl.kernel,
        mesh=plsc.VectorSubcoreMesh(core_axis_name="c", subcore_axis_name="s"),
        out_type=[
            jax.ShapeDtypeStruct((_NTOK, _D), jnp.float32),
            jax.ShapeDtypeStruct((_NW, 16), jnp.float32),
        ],
        scratch_types=[
            pltpu.VMEM((_BPW,), jnp.int32),
            pltpu.VMEM((_BPW, _D), jnp.float32),
            pltpu.VMEM((2, _ZCH, _D), jnp.float32),
            pltpu.VMEM((16,), jnp.float32),
            pltpu.SemaphoreType.DMA,
            pltpu.SemaphoreType.DMA,
            pltpu.SemaphoreType.DMA,
        ],
    )
    def _gather_sc(table_hbm, idx_hbm, z_hbm, out_hbm, loss_hbm,
                   idx_v, rows_v, z_v, part_v, sem_g, sem_z0, sem_z1):
        wid = lax.axis_index("s") * 2 + lax.axis_index("c")
        base = wid * _BPW
        pltpu.sync_copy(idx_hbm.at[pl.ds(base, _BPW)], idx_v)
        gather = pltpu.async_copy(table_hbm.at[idx_v], rows_v, sem_g)
        z_sems = [sem_z0, sem_z1]
        z_cp = [None, None]
        z_cp[0] = pltpu.async_copy(
            z_hbm.at[pl.ds(base, _ZCH)], z_v.at[0], z_sems[0])
        gather.wait()

        accs = tuple(jnp.zeros((16,), jnp.float32) for _ in range(_NLV))
        for ch in range(_NZC):
            nxt = (ch + 1) % 2
            if ch + 1 < _NZC:
                z_cp[nxt] = pltpu.async_copy(
                    z_hbm.at[pl.ds(base + (ch + 1) * _ZCH, _ZCH)],
                    z_v.at[nxt], z_sems[nxt])
            z_cp[ch % 2].wait()

            def row_body(r, accs, ch=ch):
                rr = ch * _ZCH + r
                new = []
                for c in range(_NLV):
                    q = rows_v[rr, pl.ds(c * 16, 16)]
                    zz = z_v[ch % 2, r, pl.ds(c * 16, 16)]
                    diff = q - zz
                    rows_v[rr, pl.ds(c * 16, 16)] = zz + diff
                    new.append(accs[c] + diff * diff)
                return tuple(new)

            accs = lax.fori_loop(0, _ZCH, row_body, accs)

        part = accs[0]
        for c in range(1, _NLV):
            part = part + accs[c]
        part_v[...] = part
        pltpu.sync_copy(part_v, loss_hbm.at[wid])
        pltpu.sync_copy(rows_v, out_hbm.at[pl.ds(base, _BPW)])

    return _gather_sc


def _gather_st_loss(qcb, idx, z_flat):
    return _make_gather_sc()(qcb, idx, z_flat)


def kernel(z, emb_w, proj_w, proj_b, scale):
    z_flat = z.reshape(_NTOK, _D)
    idx, qcb = _argmin_distances(z_flat, emb_w, proj_w, proj_b, scale)
    out, lparts = _gather_st_loss(qcb, idx, z_flat)
    vq_loss = jnp.sum(lparts) * jnp.float32(1.25 / (_NTOK * _D))
    return (out.reshape(_B, _T, _D), vq_loss, idx.reshape(_B, _T))


# speedup vs baseline: 1.0994x; 1.0994x over previous
import functools

import jax
import jax.numpy as jnp
from jax import lax
from jax.experimental import pallas as pl
from jax.experimental.pallas import tpu as pltpu
from jax.experimental.pallas import tpu_sc as plsc

_B, _T, _D, _K = 8, 1024, 256, 8192
_NTOK = _B * _T
_TT = 256
_NT = _NTOK // _TT
_CBT = 1024
_NW = 32
_BPW = _NTOK // _NW


_ATT = 512
_ANT = _NTOK // _ATT


def _argmin_body(scale_ref, z_ref, emb_ref, pw_ref, pb_ref,
                 idx_ref, qcb_ref, cbn_ref):
    i = pl.program_id(0)

    @pl.when(i == 0)
    def _():
        for t in range(_K // _CBT):
            q = lax.dot_general(emb_ref[pl.ds(t * _CBT, _CBT), :],
                                pw_ref[...], (((1,), (1,)), ((), ())),
                                preferred_element_type=jnp.float32)
            q = q + pb_ref[...]
            nrm = jnp.sqrt(jnp.sum(q * q, axis=-1, keepdims=True))
            qcb_ref[pl.ds(t * _CBT, _CBT), :] = q
            cbn_ref[pl.ds(t * _CBT, _CBT), :] = q / jnp.maximum(nrm, 1e-12)

    zt = z_ref[...]
    nrm = jnp.sqrt(jnp.sum(zt * zt, axis=-1, keepdims=True))
    zn = zt / jnp.maximum(nrm, 1e-12)
    s = lax.dot_general(zn, cbn_ref[...],
                        (((1,), (1,)), ((), ())),
                        preferred_element_type=jnp.float32)
    d = s * (-scale_ref[0])
    idx_ref[0, 0, :] = jnp.argmin(d, axis=1).astype(jnp.int32)


def _argmin_distances(z_flat, emb_w, proj_w, proj_b, scale):
    idx3, qcb = pl.pallas_call(
        _argmin_body,
        grid=(_ANT,),
        in_specs=[
            pl.BlockSpec(memory_space=pltpu.SMEM),
            pl.BlockSpec((_ATT, _D), lambda i: (i, 0)),
            pl.BlockSpec((_K, _D), lambda i: (0, 0)),
            pl.BlockSpec((_D, _D), lambda i: (0, 0)),
            pl.BlockSpec((1, _D), lambda i: (0, 0)),
        ],
        out_specs=[
            pl.BlockSpec((1, 1, _ATT), lambda i: (i, 0, 0)),
            pl.BlockSpec((_K, _D), lambda i: (0, 0)),
        ],
        out_shape=[
            jax.ShapeDtypeStruct((_ANT, 1, _ATT), jnp.int32),
            jax.ShapeDtypeStruct((_K, _D), jnp.float32),
        ],
        scratch_shapes=[pltpu.VMEM((_K, _D), jnp.float32)],
    )(scale.reshape(1), z_flat, emb_w, proj_w, proj_b.reshape(1, _D))
    return idx3.reshape(_NTOK), qcb


_ZCH = 64
_NZC = _BPW // _ZCH
_NLV = _D // 16


@functools.cache
def _make_gather_sc():
    @functools.partial(
        pl.kernel,
        mesh=plsc.VectorSubcoreMesh(core_axis_name="c", subcore_axis_name="s"),
        out_type=[
            jax.ShapeDtypeStruct((_NTOK, _D), jnp.float32),
            jax.ShapeDtypeStruct((_NW, 16), jnp.float32),
        ],
        scratch_types=[
            pltpu.VMEM((_BPW,), jnp.int32),
            pltpu.VMEM((_BPW, _D), jnp.float32),
            pltpu.VMEM((2, _ZCH, _D), jnp.float32),
            pltpu.VMEM((16,), jnp.float32),
            pltpu.SemaphoreType.DMA,
            pltpu.SemaphoreType.DMA,
            pltpu.SemaphoreType.DMA,
        ],
    )
    def _gather_sc(table_hbm, idx_hbm, z_hbm, out_hbm, loss_hbm,
                   idx_v, rows_v, z_v, part_v, sem_g, sem_z0, sem_z1):
        wid = lax.axis_index("s") * 2 + lax.axis_index("c")
        base = wid * _BPW
        pltpu.sync_copy(idx_hbm.at[pl.ds(base, _BPW)], idx_v)
        gather = pltpu.async_copy(table_hbm.at[idx_v], rows_v, sem_g)
        z_sems = [sem_z0, sem_z1]
        z_cp = [None, None]
        z_cp[0] = pltpu.async_copy(
            z_hbm.at[pl.ds(base, _ZCH)], z_v.at[0], z_sems[0])
        gather.wait()

        accs = tuple(jnp.zeros((16,), jnp.float32) for _ in range(_NLV))
        for ch in range(_NZC):
            nxt = (ch + 1) % 2
            if ch + 1 < _NZC:
                z_cp[nxt] = pltpu.async_copy(
                    z_hbm.at[pl.ds(base + (ch + 1) * _ZCH, _ZCH)],
                    z_v.at[nxt], z_sems[nxt])
            z_cp[ch % 2].wait()

            def row_body(r, accs, ch=ch):
                rr = ch * _ZCH + r
                new = []
                for c in range(_NLV):
                    q = rows_v[rr, pl.ds(c * 16, 16)]
                    zz = z_v[ch % 2, r, pl.ds(c * 16, 16)]
                    diff = q - zz
                    rows_v[rr, pl.ds(c * 16, 16)] = zz + diff
                    new.append(accs[c] + diff * diff)
                return tuple(new)

            accs = lax.fori_loop(0, _ZCH, row_body, accs)

        part = accs[0]
        for c in range(1, _NLV):
            part = part + accs[c]
        part_v[...] = part
        pltpu.sync_copy(part_v, loss_hbm.at[wid])
        pltpu.sync_copy(rows_v, out_hbm.at[pl.ds(base, _BPW)])

    return _gather_sc


def _gather_st_loss(qcb, idx, z_flat):
    return _make_gather_sc()(qcb, idx, z_flat)


def kernel(z, emb_w, proj_w, proj_b, scale):
    z_flat = z.reshape(_NTOK, _D)
    idx, qcb = _argmin_distances(z_flat, emb_w, proj_w, proj_b, scale)
    out, lparts = _gather_st_loss(qcb, idx, z_flat)
    vq_loss = jnp.sum(lparts) * jnp.float32(1.25 / (_NTOK * _D))
    return (out.reshape(_B, _T, _D), vq_loss, idx.reshape(_B, _T))


# speedup vs baseline: 1.1076x; 1.0075x over previous
import functools

import jax
import jax.numpy as jnp
from jax import lax
from jax.experimental import pallas as pl
from jax.experimental.pallas import tpu as pltpu
from jax.experimental.pallas import tpu_sc as plsc

_B, _T, _D, _K = 8, 1024, 256, 8192
_NTOK = _B * _T
_TT = 256
_NT = _NTOK // _TT
_CBT = 1024
_NW = 32
_BPW = _NTOK // _NW


_ATT = 1024
_ANT = _NTOK // _ATT


def _argmin_body(scale_ref, z_ref, emb_ref, pw_ref, pb_ref,
                 idx_ref, qcb_ref, cbn_ref):
    i = pl.program_id(0)

    @pl.when(i == 0)
    def _():
        for t in range(_K // _CBT):
            q = lax.dot_general(emb_ref[pl.ds(t * _CBT, _CBT), :],
                                pw_ref[...], (((1,), (1,)), ((), ())),
                                preferred_element_type=jnp.float32)
            q = q + pb_ref[...]
            nrm = jnp.sqrt(jnp.sum(q * q, axis=-1, keepdims=True))
            qcb_ref[pl.ds(t * _CBT, _CBT), :] = q
            cbn_ref[pl.ds(t * _CBT, _CBT), :] = q / jnp.maximum(nrm, 1e-12)

    zt = z_ref[...]
    nrm = jnp.sqrt(jnp.sum(zt * zt, axis=-1, keepdims=True))
    zn = zt / jnp.maximum(nrm, 1e-12)
    s = lax.dot_general(zn, cbn_ref[...],
                        (((1,), (1,)), ((), ())),
                        preferred_element_type=jnp.float32)
    d = s * (-scale_ref[0])
    idx_ref[0, 0, :] = jnp.argmin(d, axis=1).astype(jnp.int32)


def _argmin_distances(z_flat, emb_w, proj_w, proj_b, scale):
    idx3, qcb = pl.pallas_call(
        _argmin_body,
        grid=(_ANT,),
        in_specs=[
            pl.BlockSpec(memory_space=pltpu.SMEM),
            pl.BlockSpec((_ATT, _D), lambda i: (i, 0)),
            pl.BlockSpec((_K, _D), lambda i: (0, 0)),
            pl.BlockSpec((_D, _D), lambda i: (0, 0)),
            pl.BlockSpec((1, _D), lambda i: (0, 0)),
        ],
        out_specs=[
            pl.BlockSpec((1, 1, _ATT), lambda i: (i, 0, 0)),
            pl.BlockSpec((_K, _D), lambda i: (0, 0)),
        ],
        out_shape=[
            jax.ShapeDtypeStruct((_ANT, 1, _ATT), jnp.int32),
            jax.ShapeDtypeStruct((_K, _D), jnp.float32),
        ],
        scratch_shapes=[pltpu.VMEM((_K, _D), jnp.float32)],
    )(scale.reshape(1), z_flat, emb_w, proj_w, proj_b.reshape(1, _D))
    return idx3.reshape(_NTOK), qcb


_ZCH = 64
_NZC = _BPW // _ZCH
_NLV = _D // 16


@functools.cache
def _make_gather_sc():
    @functools.partial(
        pl.kernel,
        mesh=plsc.VectorSubcoreMesh(core_axis_name="c", subcore_axis_name="s"),
        out_type=[
            jax.ShapeDtypeStruct((_NTOK, _D), jnp.float32),
            jax.ShapeDtypeStruct((_NW, 16), jnp.float32),
        ],
        scratch_types=[
            pltpu.VMEM((_BPW,), jnp.int32),
            pltpu.VMEM((_BPW, _D), jnp.float32),
            pltpu.VMEM((2, _ZCH, _D), jnp.float32),
            pltpu.VMEM((16,), jnp.float32),
            pltpu.SemaphoreType.DMA,
            pltpu.SemaphoreType.DMA,
            pltpu.SemaphoreType.DMA,
        ],
    )
    def _gather_sc(table_hbm, idx_hbm, z_hbm, out_hbm, loss_hbm,
                   idx_v, rows_v, z_v, part_v, sem_g, sem_z0, sem_z1):
        wid = lax.axis_index("s") * 2 + lax.axis_index("c")
        base = wid * _BPW
        pltpu.sync_copy(idx_hbm.at[pl.ds(base, _BPW)], idx_v)
        gather = pltpu.async_copy(table_hbm.at[idx_v], rows_v, sem_g)
        z_sems = [sem_z0, sem_z1]
        z_cp = [None, None]
        z_cp[0] = pltpu.async_copy(
            z_hbm.at[pl.ds(base, _ZCH)], z_v.at[0], z_sems[0])
        gather.wait()

        accs = tuple(jnp.zeros((16,), jnp.float32) for _ in range(_NLV))
        for ch in range(_NZC):
            nxt = (ch + 1) % 2
            if ch + 1 < _NZC:
                z_cp[nxt] = pltpu.async_copy(
                    z_hbm.at[pl.ds(base + (ch + 1) * _ZCH, _ZCH)],
                    z_v.at[nxt], z_sems[nxt])
            z_cp[ch % 2].wait()

            def row_body(r, accs, ch=ch):
                rr = ch * _ZCH + r
                new = []
                for c in range(_NLV):
                    q = rows_v[rr, pl.ds(c * 16, 16)]
                    zz = z_v[ch % 2, r, pl.ds(c * 16, 16)]
                    diff = q - zz
                    rows_v[rr, pl.ds(c * 16, 16)] = zz + diff
                    new.append(accs[c] + diff * diff)
                return tuple(new)

            accs = lax.fori_loop(0, _ZCH, row_body, accs)

        part = accs[0]
        for c in range(1, _NLV):
            part = part + accs[c]
        part_v[...] = part
        pltpu.sync_copy(part_v, loss_hbm.at[wid])
        pltpu.sync_copy(rows_v, out_hbm.at[pl.ds(base, _BPW)])

    return _gather_sc


def _gather_st_loss(qcb, idx, z_flat):
    return _make_gather_sc()(qcb, idx, z_flat)


def kernel(z, emb_w, proj_w, proj_b, scale):
    z_flat = z.reshape(_NTOK, _D)
    idx, qcb = _argmin_distances(z_flat, emb_w, proj_w, proj_b, scale)
    out, lparts = _gather_st_loss(qcb, idx, z_flat)
    vq_loss = jnp.sum(lparts) * jnp.float32(1.25 / (_NTOK * _D))
    return (out.reshape(_B, _T, _D), vq_loss, idx.reshape(_B, _T))


# speedup vs baseline: 1.1115x; 1.0035x over previous
import functools

import jax
import jax.numpy as jnp
from jax import lax
from jax.experimental import pallas as pl
from jax.experimental.pallas import tpu as pltpu
from jax.experimental.pallas import tpu_sc as plsc

_B, _T, _D, _K = 8, 1024, 256, 8192
_NTOK = _B * _T
_TT = 256
_NT = _NTOK // _TT
_CBT = 1024
_NW = 32
_BPW = _NTOK // _NW


_ATT = 1024
_ANT = _NTOK // _ATT


def _argmin_body(scale_ref, z_ref, emb_ref, pw_ref, pb_ref,
                 idx_ref, qcb_ref, cbn_ref):
    i = pl.program_id(0)

    @pl.when(i == 0)
    def _():
        for t in range(_K // _CBT):
            q = lax.dot_general(emb_ref[pl.ds(t * _CBT, _CBT), :],
                                pw_ref[...], (((1,), (1,)), ((), ())),
                                preferred_element_type=jnp.float32)
            q = q + pb_ref[...]
            nrm = jnp.sqrt(jnp.sum(q * q, axis=-1, keepdims=True))
            qcb_ref[pl.ds(t * _CBT, _CBT), :] = q
            cbn_ref[pl.ds(t * _CBT, _CBT), :] = q / jnp.maximum(nrm, 1e-12)

    zt = z_ref[...]
    nrm = jnp.sqrt(jnp.sum(zt * zt, axis=-1, keepdims=True))
    zn = zt / jnp.maximum(nrm, 1e-12)
    s = lax.dot_general(zn, cbn_ref[...],
                        (((1,), (1,)), ((), ())),
                        preferred_element_type=jnp.float32)
    d = s * (-scale_ref[0])
    idx_ref[0, 0, :] = jnp.argmin(d, axis=1).astype(jnp.int32)


def _argmin_distances(z_flat, emb_w, proj_w, proj_b, scale):
    idx3, qcb = pl.pallas_call(
        _argmin_body,
        grid=(_ANT,),
        in_specs=[
            pl.BlockSpec(memory_space=pltpu.SMEM),
            pl.BlockSpec((_ATT, _D), lambda i: (i, 0)),
            pl.BlockSpec((_K, _D), lambda i: (0, 0)),
            pl.BlockSpec((_D, _D), lambda i: (0, 0)),
            pl.BlockSpec((1, _D), lambda i: (0, 0)),
        ],
        out_specs=[
            pl.BlockSpec((1, 1, _ATT), lambda i: (i, 0, 0)),
            pl.BlockSpec((_K, _D), lambda i: (0, 0)),
        ],
        out_shape=[
            jax.ShapeDtypeStruct((_ANT, 1, _ATT), jnp.int32),
            jax.ShapeDtypeStruct((_K, _D), jnp.float32),
        ],
        scratch_shapes=[pltpu.VMEM((_K, _D), jnp.float32)],
    )(scale.reshape(1), z_flat, emb_w, proj_w, proj_b.reshape(1, _D))
    return idx3.reshape(_NTOK), qcb


_ZCH = 64
_NZC = _BPW // _ZCH
_NLV = _D // 16


@functools.cache
def _make_gather_sc():
    @functools.partial(
        pl.kernel,
        mesh=plsc.VectorSubcoreMesh(core_axis_name="c", subcore_axis_name="s"),
        out_type=[
            jax.ShapeDtypeStruct((_NTOK, _D), jnp.float32),
            jax.ShapeDtypeStruct((_NW, 16), jnp.float32),
        ],
        scratch_types=[
            pltpu.VMEM((_BPW,), jnp.int32),
            pltpu.VMEM((_BPW, _D), jnp.float32),
            pltpu.VMEM((2, _ZCH, _D), jnp.float32),
            pltpu.VMEM((16,), jnp.float32),
            pltpu.SemaphoreType.DMA,
            pltpu.SemaphoreType.DMA,
            pltpu.SemaphoreType.DMA,
            pltpu.SemaphoreType.DMA,
            pltpu.SemaphoreType.DMA,
        ],
    )
    def _gather_sc(table_hbm, idx_hbm, z_hbm, out_hbm, loss_hbm,
                   idx_v, rows_v, z_v, part_v,
                   sem_g0, sem_g1, sem_z0, sem_z1, sem_o):
        wid = lax.axis_index("s") * 2 + lax.axis_index("c")
        base = wid * _BPW
        pltpu.sync_copy(idx_hbm.at[pl.ds(base, _BPW)], idx_v)
        g_sems = [sem_g0, sem_g1]
        z_sems = [sem_z0, sem_z1]
        g_cp = [None, None]
        z_cp = [None, None]

        def start_chunk(ch):
            b = ch % 2
            g_cp[b] = pltpu.async_copy(
                table_hbm.at[idx_v.at[pl.ds(ch * _ZCH, _ZCH)]],
                rows_v.at[pl.ds(ch * _ZCH, _ZCH)], g_sems[b])
            z_cp[b] = pltpu.async_copy(
                z_hbm.at[pl.ds(base + ch * _ZCH, _ZCH)],
                z_v.at[b], z_sems[b])

        start_chunk(0)
        out_cps = []
        accs = tuple(jnp.zeros((16,), jnp.float32) for _ in range(_NLV))
        for ch in range(_NZC):
            b = ch % 2
            if ch + 1 < _NZC:
                start_chunk(ch + 1)
            g_cp[b].wait()
            z_cp[b].wait()

            def row_body(r, accs, ch=ch, b=b):
                rr = ch * _ZCH + r
                new = []
                for c in range(_NLV):
                    q = rows_v[rr, pl.ds(c * 16, 16)]
                    zz = z_v[b, r, pl.ds(c * 16, 16)]
                    diff = q - zz
                    rows_v[rr, pl.ds(c * 16, 16)] = zz + diff
                    new.append(accs[c] + diff * diff)
                return tuple(new)

            accs = lax.fori_loop(0, _ZCH, row_body, accs)
            out_cps.append(pltpu.async_copy(
                rows_v.at[pl.ds(ch * _ZCH, _ZCH)],
                out_hbm.at[pl.ds(base + ch * _ZCH, _ZCH)], sem_o))

        part = accs[0]
        for c in range(1, _NLV):
            part = part + accs[c]
        part_v[...] = part
        pltpu.sync_copy(part_v, loss_hbm.at[wid])
        for cp in out_cps:
            cp.wait()

    return _gather_sc


def _gather_st_loss(qcb, idx, z_flat):
    return _make_gather_sc()(qcb, idx, z_flat)


def kernel(z, emb_w, proj_w, proj_b, scale):
    z_flat = z.reshape(_NTOK, _D)
    idx, qcb = _argmin_distances(z_flat, emb_w, proj_w, proj_b, scale)
    out, lparts = _gather_st_loss(qcb, idx, z_flat)
    vq_loss = jnp.sum(lparts) * jnp.float32(1.25 / (_NTOK * _D))
    return (out.reshape(_B, _T, _D), vq_loss, idx.reshape(_B, _T))
